# baseline (speedup 1.0000x reference)
import jax
import jax.numpy as jnp
from jax import lax
from jax.experimental import pallas as pl
from jax.experimental.pallas import tpu as pltpu
from jax.experimental.pallas import tpu_sc as plsc

NC, NS, L = 2, 16, 16
NW = NC * NS
B = 16384
D = 64
BPW = B // NW
CH = 128
NCH = BPW // CH


def _body(hidx_hbm, ridx_hbm, tidx_hbm, ent_hbm, rel_hbm, out_hbm,
          hidx_v, ridx_v, tidx_v, hrows, rrows, trows, score_v, tpose_v,
          sem_h, sem_r, sem_t):
    wid = lax.axis_index("s") * NC + lax.axis_index("c")
    pltpu.sync_copy(hidx_hbm.at[wid], hidx_v)
    pltpu.sync_copy(ridx_hbm.at[wid], ridx_v)
    pltpu.sync_copy(tidx_hbm.at[wid], tidx_v)
    for j in range(NCH):
        cph = pltpu.async_copy(ent_hbm.at[hidx_v.at[j]], hrows, sem_h)
        cpr = pltpu.async_copy(rel_hbm.at[ridx_v.at[j]], rrows, sem_r)
        cpt = pltpu.async_copy(ent_hbm.at[tidx_v.at[j]], trows, sem_t)
        cph.wait()
        cpr.wait()
        cpt.wait()

        lane = lax.iota(jnp.int32, L)

        def compute(g, carry):
            for ii in range(L):
                i = g * L + ii
                acc = None
                for c in range(D // L):
                    h = hrows[i, pl.ds(c * L, L)]
                    r = rrows[i, pl.ds(c * L, L)]
                    t = trows[i, pl.ds(c * L, L)]
                    a = jnp.abs(h + r - t)
                    acc = a if acc is None else acc + a
                tpose_v[ii, pl.ds(0, L)] = acc
            svec = None
            for c in range(L):
                part = plsc.load_gather(
                    tpose_v, [lane, jnp.full((L,), c, jnp.int32)])
                svec = part if svec is None else svec + part
            score_v[pl.ds(g * L, L)] = svec
            return carry

        lax.fori_loop(0, CH // L, compute, 0)
        base = wid * BPW + j * CH
        pltpu.sync_copy(score_v, out_hbm.at[pl.ds(base, CH)])


def kernel(sample, entity_embedding, relation_embedding):
    hidx = sample[:, 0].reshape(NW, NCH, CH)
    ridx = sample[:, 1].reshape(NW, NCH, CH)
    tidx = sample[:, 2].reshape(NW, NCH, CH)
    mesh = plsc.VectorSubcoreMesh(
        core_axis_name="c", subcore_axis_name="s",
        num_cores=NC, num_subcores=NS)
    f = pl.kernel(
        _body,
        out_type=jax.ShapeDtypeStruct((B,), jnp.float32),
        mesh=mesh,
        compiler_params=pltpu.CompilerParams(
            needs_layout_passes=False, use_tc_tiling_on_sc=False),
        scratch_types=[
            pltpu.VMEM((NCH, CH), jnp.int32),
            pltpu.VMEM((NCH, CH), jnp.int32),
            pltpu.VMEM((NCH, CH), jnp.int32),
            pltpu.VMEM((CH, D), jnp.float32),
            pltpu.VMEM((CH, D), jnp.float32),
            pltpu.VMEM((CH, D), jnp.float32),
            pltpu.VMEM((CH,), jnp.float32),
            pltpu.VMEM((L, L + 1), jnp.float32),
            pltpu.SemaphoreType.DMA,
            pltpu.SemaphoreType.DMA,
            pltpu.SemaphoreType.DMA,
        ],
    )
    score = f(hidx, ridx, tidx, entity_embedding, relation_embedding)
    return score.reshape(B, 1)
